# baseline (speedup 1.0000x reference)
import functools

import jax
import jax.numpy as jnp
from jax import lax
from jax.experimental import pallas as pl
from jax.experimental.pallas import tpu as pltpu
from jax.experimental.pallas import tpu_sc as plsc

_N = 10000
_NP = 10240
_E = 320000
_EP = 327680
_D = 128
_ROWS = _EP // 128
_CHUNK = 128
_SLAB = 8
_NC = 2
_NS = 16
_L = 16
_NPT = _NP // _NS
_NPA = 10008
_BLK = 2048
_GRID = _NP // _BLK


def _sc_body(x_hbm, src_hbm, dst_hbm, ones_hbm, zrow_hbm, zacc_hbm,
             acc_out, c_out, inv_out,
             dslabA, dslabB, sslabA, sslabB,
             rows1, rows2, wbufA, wbufB,
             ones_v, invl, nbuf,
             gsem1, gsem2, ssem1, ssem2,
             csemA, csemB, dsemA, dsemB, asem,
             cntS, cS, accS):
  cid = lax.axis_index("c")
  sid = lax.axis_index("s")
  wid = cid * _NS + sid
  chunk0 = sid * _NPT
  dbuf = [dslabA, dslabB]
  sbuf = [sslabA, sslabB]
  rbuf = [rows1, rows2]
  wbuf = [wbufA, wbufB]
  gsem = [gsem1, gsem2]
  ssem = [ssem1, ssem2]
  csem = [csemA, csemB]
  dsem = [dsemA, dsemB]

  scope = jax.named_scope
  with scope("z0"):
    pltpu.sync_copy(zrow_hbm, cntS.at[pl.ds(chunk0, _NPT)])
  pltpu.sync_copy(zrow_hbm, cS.at[pl.ds(chunk0, _NPT)])
  @pl.when(sid < _NS - 1)
  def _():
    pltpu.sync_copy(zacc_hbm, accS.at[pl.ds(chunk0, _NPT)])

  @pl.when(sid == _NS - 1)
  def _():
    pltpu.sync_copy(zacc_hbm.at[pl.ds(0, _NPA - (_NS - 1) * _NPT)],
                    accS.at[pl.ds((_NS - 1) * _NPT, _NPA - (_NS - 1) * _NPT)])
  pltpu.sync_copy(ones_hbm, ones_v)
  plsc.subcore_barrier()

  rows_p1 = _ROWS // _NS
  nslab1 = rows_p1 // _SLAB
  p1base = sid * rows_p1
  p1ctx = scope("p1_counts"); p1ctx.__enter__()
  dmas = [None] * nslab1
  adds = [[] for _ in range(nslab1)]
  dmas[0] = pltpu.async_copy(
      dst_hbm.at[pl.ds(p1base, _SLAB)], dbuf[0], dsem[0])
  for t in range(nslab1):
    if t >= 1:
      for d in adds[t - 1]:
        d.wait()
    dmas[t].wait()
    if t + 1 < nslab1:
      dmas[t + 1] = pltpu.async_copy(
          dst_hbm.at[pl.ds(p1base + (t + 1) * _SLAB, _SLAB)],
          dbuf[(t + 1) % 2], dsem[(t + 1) % 2])
    for j in range(_SLAB):
      adds[t].append(pltpu.async_copy(
          ones_v, cntS.at[dbuf[t % 2].at[j]], asem, add=True))
  for d in adds[nslab1 - 1]:
    d.wait()
  plsc.subcore_barrier()
  p1ctx.__exit__(None, None, None)

  p1bctx = scope("p1b_inv"); p1bctx.__enter__()
  pltpu.sync_copy(cntS.at[pl.ds(chunk0, _NPT)], nbuf)

  def inv_vec(i, carry):
    v = nbuf[pl.ds(i * _L, _L)]
    nbuf[pl.ds(i * _L, _L)] = 1.0 / jnp.maximum(v, 1.0)
    return carry

  lax.fori_loop(0, _NPT // _L, inv_vec, 0)
  pltpu.sync_copy(nbuf, cntS.at[pl.ds(chunk0, _NPT)])

  @pl.when(cid == 0)
  def _():
    pltpu.sync_copy(nbuf, inv_out.at[pl.ds(chunk0, _NPT)])

  plsc.subcore_barrier()
  pltpu.sync_copy(cntS, invl)
  p1bctx.__exit__(None, None, None)

  rows_pt = _ROWS // (_NC * _NS)
  nslab23 = rows_pt // _SLAB
  base0 = wid * rows_pt

  def load_slab(t):
    b = pl.ds(base0 + t * _SLAB, _SLAB)
    return (pltpu.async_copy(src_hbm.at[b], sbuf[t % 2], dsem[t % 2]),
            pltpu.async_copy(dst_hbm.at[b], dbuf[t % 2], asem))

  p23ctx = scope("p23_rows"); p23ctx.__enter__()
  nch = rows_pt
  sl = [None] * nslab23
  gd = [None] * nch
  sd = [None] * nch
  cd = [None] * nch
  _NB = 2
  _AH = 1
  sl[0] = load_slab(0)
  for d in sl[0]:
    d.wait()
  for f in range(_AH):
    gd[f] = pltpu.async_copy(
        x_hbm.at[sbuf[0].at[f]], rbuf[f % _NB], gsem[f % _NB])
  for g in range(nch):
    t, j = divmod(g, _SLAB)
    f = g + _AH
    if f < nch:
      tf, jf = divmod(f, _SLAB)
      if jf == 0:
        for d in sl[tf]:
          d.wait()
      if f - _NB >= 0:
        sd[f - _NB].wait()
      gd[f] = pltpu.async_copy(
          x_hbm.at[sbuf[tf % 2].at[jf]], rbuf[f % _NB], gsem[f % _NB])
    if j == 2 and t + 1 < nslab23:
      sl[t + 1] = load_slab(t + 1)
    if g >= 2:
      cd[g - 2].wait()
    for k in range(_CHUNK // _L):
      d = dbuf[t % 2][j, pl.ds(k * _L, _L)]
      wbuf[g % 2][pl.ds(k * _L, _L)] = plsc.load_gather(invl, [d])
    cd[g] = pltpu.async_copy(
        wbuf[g % 2], cS.at[sbuf[t % 2].at[j]], csem[g % 2], add=True)
    gd[g].wait()
    sd[g] = pltpu.async_copy(
        rbuf[g % _NB], accS.at[dbuf[t % 2].at[j]], ssem[g % _NB], add=True)
  for k in range(nch - _NB + _AH, nch):
    sd[k].wait()
  cd[nch - 2].wait()
  cd[nch - 1].wait()
  plsc.subcore_barrier()
  p23ctx.__exit__(None, None, None)

  @pl.when(sid < _NS - 1)
  def _():
    pltpu.sync_copy(accS.at[pl.ds(chunk0, _NPT)],
                    acc_out.at[cid, pl.ds(chunk0, _NPT)])

  @pl.when(sid == _NS - 1)
  def _():
    tail = _NPA - (_NS - 1) * _NPT
    pltpu.sync_copy(accS.at[pl.ds((_NS - 1) * _NPT, tail)],
                    acc_out.at[cid, pl.ds((_NS - 1) * _NPT, tail)])
  pltpu.sync_copy(cS.at[pl.ds(chunk0, _NPT)],
                  c_out.at[cid, pl.ds(chunk0, _NPT)])


_sc_graph = functools.partial(
    pl.kernel,
    out_type=[
        jax.ShapeDtypeStruct((_NC, _NP, _D), jnp.float32),
        jax.ShapeDtypeStruct((_NC, _NP), jnp.float32),
        jax.ShapeDtypeStruct((_NP,), jnp.float32),
    ],
    mesh=plsc.VectorSubcoreMesh(
        core_axis_name="c", subcore_axis_name="s",
        num_cores=_NC, num_subcores=_NS),
    compiler_params=pltpu.CompilerParams(needs_layout_passes=False),
    scratch_types=[
        pltpu.VMEM((_SLAB, _CHUNK), jnp.int32),
        pltpu.VMEM((_SLAB, _CHUNK), jnp.int32),
        pltpu.VMEM((_SLAB, _CHUNK), jnp.int32),
        pltpu.VMEM((_SLAB, _CHUNK), jnp.int32),
        pltpu.VMEM((_CHUNK, _D), jnp.float32),
        pltpu.VMEM((_CHUNK, _D), jnp.float32),
        pltpu.VMEM((_CHUNK,), jnp.float32),
        pltpu.VMEM((_CHUNK,), jnp.float32),
        pltpu.VMEM((_CHUNK,), jnp.float32),
        pltpu.VMEM((_NP,), jnp.float32),
        pltpu.VMEM((_NPT,), jnp.float32),
        pltpu.SemaphoreType.DMA,
        pltpu.SemaphoreType.DMA,
        pltpu.SemaphoreType.DMA,
        pltpu.SemaphoreType.DMA,
        pltpu.SemaphoreType.DMA,
        pltpu.SemaphoreType.DMA,
        pltpu.SemaphoreType.DMA,
        pltpu.SemaphoreType.DMA,
        pltpu.SemaphoreType.DMA,
        pltpu.VMEM_SHARED((_NP,), jnp.float32),
        pltpu.VMEM_SHARED((_NP,), jnp.float32),
        pltpu.VMEM_SHARED((_NPA, _D), jnp.float32),
    ],
)(_sc_body)


def _tc_body(x_ref, acc_ref, inv_ref, c_ref,
             w1l_ref, w1r_ref, b1l_ref, w2l_ref, b2l_ref, w2r_ref,
             out_ref, sh_ref, sc_ref):
  i = pl.program_id(0)
  rows = acc_ref[0] + acc_ref[1]
  agg = rows * inv_ref[...]
  pre = jnp.dot(agg, w1l_ref[...], preferred_element_type=jnp.float32)
  pre += jnp.dot(x_ref[...], w1r_ref[...], preferred_element_type=jnp.float32)
  pre += b1l_ref[...]
  h = jnp.maximum(pre, 0.0)
  rid = lax.broadcasted_iota(jnp.int32, (_BLK, 1), 0) + i * _BLK
  valid = rid < _N
  h = jnp.where(valid, h, 0.0)
  cv = jnp.where(valid, c_ref[0] + c_ref[1], 0.0)

  @pl.when(i == 0)
  def _():
    sh_ref[...] = jnp.zeros_like(sh_ref)
    sc_ref[...] = jnp.zeros_like(sc_ref)

  sh_ref[...] += jnp.sum(h, axis=0, keepdims=True)
  sc_ref[...] += jnp.sum(h * cv, axis=0, keepdims=True)

  @pl.when(i == pl.num_programs(0) - 1)
  def _():
    mh = sh_ref[...] * (1.0 / _N)
    mc = sc_ref[...] * (1.0 / _N)
    out_ref[...] = (
        jnp.dot(mc, w2l_ref[...], preferred_element_type=jnp.float32)
        + b2l_ref[...]
        + jnp.dot(mh, w2r_ref[...], preferred_element_type=jnp.float32))


def _tc_dense(x_pad, acc, inv, cpart, W1l, W1r, b1l, W2l, b2l, W2r):
  full = lambda shape: pl.BlockSpec(shape, lambda i: (0,) * len(shape))
  return pl.pallas_call(
      _tc_body,
      grid=(_GRID,),
      in_specs=[
          pl.BlockSpec((_BLK, _D), lambda i: (i, 0)),
          pl.BlockSpec((_NC, _BLK, _D), lambda i: (0, i, 0)),
          pl.BlockSpec((_BLK, 1), lambda i: (i, 0)),
          pl.BlockSpec((_NC, _BLK, 1), lambda i: (0, i, 0)),
          full((_D, _D)),
          full((_D, _D)),
          full((1, _D)),
          full((_D, _D)),
          full((1, _D)),
          full((_D, _D)),
      ],
      out_specs=pl.BlockSpec((1, _D), lambda i: (0, 0)),
      out_shape=jax.ShapeDtypeStruct((1, _D), jnp.float32),
      scratch_shapes=[
          pltpu.VMEM((1, _D), jnp.float32),
          pltpu.VMEM((1, _D), jnp.float32),
      ],
      compiler_params=pltpu.CompilerParams(
          dimension_semantics=("arbitrary",)),
  )(x_pad, acc, inv, cpart, W1l, W1r, b1l, W2l, b2l, W2r)


def kernel(x, edge_index, W1l, b1l, W1r, W2l, b2l, W2r):
  x_pad = jnp.concatenate(
      [x, jnp.zeros((_NP - _N, _D), jnp.float32)], axis=0)
  idx = jnp.arange(_EP - _E, dtype=jnp.int32)
  pad = jnp.stack([_N + idx % 128, _N + idx % (_NPA - _N)])
  ei = jnp.concatenate([edge_index, pad], axis=1)
  src2 = ei[0].reshape(_ROWS, _CHUNK)
  dst2 = ei[1].reshape(_ROWS, _CHUNK)
  ones_h = jnp.ones((_CHUNK,), jnp.float32)
  zrow = jnp.zeros((_NPT,), jnp.float32)
  zacc = jnp.zeros((_NPT, _D), jnp.float32)

  acc, cpart, inv = _sc_graph(x_pad, src2, dst2, ones_h, zrow, zacc)

  out = _tc_dense(x_pad, acc, inv.reshape(_NP, 1),
                  cpart.reshape(_NC, _NP, 1),
                  W1l, W1r, b1l.reshape(1, _D),
                  W2l, b2l.reshape(1, _D), W2r)
  return out.reshape(_D)


# speedup vs baseline: 1.1172x; 1.1172x over previous
import functools

import jax
import jax.numpy as jnp
from jax import lax
from jax.experimental import pallas as pl
from jax.experimental.pallas import tpu as pltpu
from jax.experimental.pallas import tpu_sc as plsc

_N = 10000
_NP = 10240
_E = 320000
_EP = 327680
_D = 128
_ROWS = _EP // 128
_CHUNK = 128
_SLAB = 8
_NC = 2
_NS = 16
_L = 16
_NPT = _NP // _NS
_NPA = 10008
_BLK = 2048
_GRID = _NP // _BLK


def _sc_body(x_hbm, src_hbm, dst_hbm, ones_hbm, zrow_hbm, zacc_hbm,
             acc_out, c_out, inv_out,
             dslabA, dslabB, sslabA, sslabB,
             rows1, rows2, wbufA, wbufB,
             ones_v, invl, nbuf,
             gsem1, gsem2, ssem1, ssem2,
             csemA, csemB, dsemA, dsemB, asem,
             cntS, cS, accS):
  cid = lax.axis_index("c")
  sid = lax.axis_index("s")
  wid = cid * _NS + sid
  chunk0 = sid * _NPT
  dbuf = [dslabA, dslabB]
  sbuf = [sslabA, sslabB]
  rbuf = [rows1, rows2]
  wbuf = [wbufA, wbufB]
  gsem = [gsem1, gsem2]
  ssem = [ssem1, ssem2]
  csem = [csemA, csemB]
  dsem = [dsemA, dsemB]

  scope = jax.named_scope
  with scope("z0"):
    pltpu.sync_copy(zrow_hbm, cntS.at[pl.ds(chunk0, _NPT)])
  pltpu.sync_copy(zrow_hbm, cS.at[pl.ds(chunk0, _NPT)])
  @pl.when(sid < _NS - 1)
  def _():
    pltpu.sync_copy(zacc_hbm, accS.at[pl.ds(chunk0, _NPT)])

  @pl.when(sid == _NS - 1)
  def _():
    pltpu.sync_copy(zacc_hbm.at[pl.ds(0, _NPA - (_NS - 1) * _NPT)],
                    accS.at[pl.ds((_NS - 1) * _NPT, _NPA - (_NS - 1) * _NPT)])
  pltpu.sync_copy(ones_hbm, ones_v)
  plsc.subcore_barrier()

  rows_p1 = _ROWS // _NS
  nslab1 = rows_p1 // _SLAB
  p1base = sid * rows_p1
  p1ctx = scope("p1_counts"); p1ctx.__enter__()
  dmas = [None] * nslab1
  adds = [[] for _ in range(nslab1)]
  dmas[0] = pltpu.async_copy(
      dst_hbm.at[pl.ds(p1base, _SLAB)], dbuf[0], dsem[0])
  for t in range(nslab1):
    if t >= 1:
      for d in adds[t - 1]:
        d.wait()
    dmas[t].wait()
    if t + 1 < nslab1:
      dmas[t + 1] = pltpu.async_copy(
          dst_hbm.at[pl.ds(p1base + (t + 1) * _SLAB, _SLAB)],
          dbuf[(t + 1) % 2], dsem[(t + 1) % 2])
    for j in range(_SLAB):
      adds[t].append(pltpu.async_copy(
          ones_v, cntS.at[dbuf[t % 2].at[j]], asem, add=True))
  for d in adds[nslab1 - 1]:
    d.wait()
  plsc.subcore_barrier()
  p1ctx.__exit__(None, None, None)

  p1bctx = scope("p1b_inv"); p1bctx.__enter__()
  pltpu.sync_copy(cntS.at[pl.ds(chunk0, _NPT)], nbuf)

  def inv_vec(i, carry):
    v = nbuf[pl.ds(i * _L, _L)]
    nbuf[pl.ds(i * _L, _L)] = 1.0 / jnp.maximum(v, 1.0)
    return carry

  lax.fori_loop(0, _NPT // _L, inv_vec, 0)
  pltpu.sync_copy(nbuf, cntS.at[pl.ds(chunk0, _NPT)])

  @pl.when(cid == 0)
  def _():
    pltpu.sync_copy(nbuf, inv_out.at[pl.ds(chunk0, _NPT)])

  plsc.subcore_barrier()
  pltpu.sync_copy(cntS, invl)
  p1bctx.__exit__(None, None, None)

  rows_pt = _ROWS // (_NC * _NS)
  nslab23 = rows_pt // _SLAB
  base0 = wid * rows_pt

  def load_slab(t):
    b = pl.ds(base0 + t * _SLAB, _SLAB)
    return (pltpu.async_copy(src_hbm.at[b], sbuf[t % 2], dsem[t % 2]),
            pltpu.async_copy(dst_hbm.at[b], dbuf[t % 2], asem))

  p23ctx = scope("p23_rows"); p23ctx.__enter__()
  nch = rows_pt
  sl = [None] * nslab23
  gd = [None] * nch
  sd = [None] * nch
  cd = [None] * nch
  sl[0] = load_slab(0)
  for g in range(nch):
    t, j = divmod(g, _SLAB)
    if j == 4 and t + 1 < nslab23:
      sl[t + 1] = load_slab(t + 1)
    if g == 0:
      for d in sl[0]:
        d.wait()
      gd[0] = pltpu.async_copy(
          x_hbm.at[sbuf[0].at[0]], rbuf[0], gsem[0])
    if g + 1 < nch:
      t1, j1 = divmod(g + 1, _SLAB)
      if j1 == 0:
        for d in sl[t1]:
          d.wait()
      if g - 1 >= 0:
        sd[g - 1].wait()
      gd[g + 1] = pltpu.async_copy(
          x_hbm.at[sbuf[t1 % 2].at[j1]], rbuf[(g + 1) % 2],
          gsem[(g + 1) % 2])
    if g >= 2:
      cd[g - 2].wait()
    for k in range(_CHUNK // _L):
      d = dbuf[t % 2][j, pl.ds(k * _L, _L)]
      wbuf[g % 2][pl.ds(k * _L, _L)] = plsc.load_gather(invl, [d])
    cd[g] = pltpu.async_copy(
        wbuf[g % 2], cS.at[sbuf[t % 2].at[j]], csem[g % 2], add=True)
    gd[g].wait()
    sd[g] = pltpu.async_copy(
        rbuf[g % 2], accS.at[dbuf[t % 2].at[j]], ssem[g % 2], add=True)
  sd[nch - 2].wait()
  sd[nch - 1].wait()
  cd[nch - 2].wait()
  cd[nch - 1].wait()
  plsc.subcore_barrier()
  p23ctx.__exit__(None, None, None)

  @pl.when(sid < _NS - 1)
  def _():
    pltpu.sync_copy(accS.at[pl.ds(chunk0, _NPT)],
                    acc_out.at[cid, pl.ds(chunk0, _NPT)])

  @pl.when(sid == _NS - 1)
  def _():
    tail = _NPA - (_NS - 1) * _NPT
    pltpu.sync_copy(accS.at[pl.ds((_NS - 1) * _NPT, tail)],
                    acc_out.at[cid, pl.ds((_NS - 1) * _NPT, tail)])
  pltpu.sync_copy(cS.at[pl.ds(chunk0, _NPT)],
                  c_out.at[cid, pl.ds(chunk0, _NPT)])


_sc_graph = functools.partial(
    pl.kernel,
    out_type=[
        jax.ShapeDtypeStruct((_NC, _NP, _D), jnp.float32),
        jax.ShapeDtypeStruct((_NC, _NP), jnp.float32),
        jax.ShapeDtypeStruct((_NP,), jnp.float32),
    ],
    mesh=plsc.VectorSubcoreMesh(
        core_axis_name="c", subcore_axis_name="s",
        num_cores=_NC, num_subcores=_NS),
    compiler_params=pltpu.CompilerParams(needs_layout_passes=False),
    scratch_types=[
        pltpu.VMEM((_SLAB, _CHUNK), jnp.int32),
        pltpu.VMEM((_SLAB, _CHUNK), jnp.int32),
        pltpu.VMEM((_SLAB, _CHUNK), jnp.int32),
        pltpu.VMEM((_SLAB, _CHUNK), jnp.int32),
        pltpu.VMEM((_CHUNK, _D), jnp.float32),
        pltpu.VMEM((_CHUNK, _D), jnp.float32),
        pltpu.VMEM((_CHUNK,), jnp.float32),
        pltpu.VMEM((_CHUNK,), jnp.float32),
        pltpu.VMEM((_CHUNK,), jnp.float32),
        pltpu.VMEM((_NP,), jnp.float32),
        pltpu.VMEM((_NPT,), jnp.float32),
        pltpu.SemaphoreType.DMA,
        pltpu.SemaphoreType.DMA,
        pltpu.SemaphoreType.DMA,
        pltpu.SemaphoreType.DMA,
        pltpu.SemaphoreType.DMA,
        pltpu.SemaphoreType.DMA,
        pltpu.SemaphoreType.DMA,
        pltpu.SemaphoreType.DMA,
        pltpu.SemaphoreType.DMA,
        pltpu.VMEM_SHARED((_NP,), jnp.float32),
        pltpu.VMEM_SHARED((_NP,), jnp.float32),
        pltpu.VMEM_SHARED((_NPA, _D), jnp.float32),
    ],
)(_sc_body)


def _tc_body(x_ref, acc_ref, inv_ref, c_ref,
             w1l_ref, w1r_ref, b1l_ref, w2l_ref, b2l_ref, w2r_ref,
             out_ref, sh_ref, sc_ref):
  i = pl.program_id(0)
  rows = acc_ref[0] + acc_ref[1]
  agg = rows * inv_ref[...]
  pre = jnp.dot(agg, w1l_ref[...], preferred_element_type=jnp.float32)
  pre += jnp.dot(x_ref[...], w1r_ref[...], preferred_element_type=jnp.float32)
  pre += b1l_ref[...]
  h = jnp.maximum(pre, 0.0)
  rid = lax.broadcasted_iota(jnp.int32, (_BLK, 1), 0) + i * _BLK
  valid = rid < _N
  h = jnp.where(valid, h, 0.0)
  cv = jnp.where(valid, c_ref[0] + c_ref[1], 0.0)

  @pl.when(i == 0)
  def _():
    sh_ref[...] = jnp.zeros_like(sh_ref)
    sc_ref[...] = jnp.zeros_like(sc_ref)

  sh_ref[...] += jnp.sum(h, axis=0, keepdims=True)
  sc_ref[...] += jnp.sum(h * cv, axis=0, keepdims=True)

  @pl.when(i == pl.num_programs(0) - 1)
  def _():
    mh = sh_ref[...] * (1.0 / _N)
    mc = sc_ref[...] * (1.0 / _N)
    out_ref[...] = (
        jnp.dot(mc, w2l_ref[...], preferred_element_type=jnp.float32)
        + b2l_ref[...]
        + jnp.dot(mh, w2r_ref[...], preferred_element_type=jnp.float32))


def _tc_dense(x_pad, acc, inv, cpart, W1l, W1r, b1l, W2l, b2l, W2r):
  full = lambda shape: pl.BlockSpec(shape, lambda i: (0,) * len(shape))
  return pl.pallas_call(
      _tc_body,
      grid=(_GRID,),
      in_specs=[
          pl.BlockSpec((_BLK, _D), lambda i: (i, 0)),
          pl.BlockSpec((_NC, _BLK, _D), lambda i: (0, i, 0)),
          pl.BlockSpec((_BLK, 1), lambda i: (i, 0)),
          pl.BlockSpec((_NC, _BLK, 1), lambda i: (0, i, 0)),
          full((_D, _D)),
          full((_D, _D)),
          full((1, _D)),
          full((_D, _D)),
          full((1, _D)),
          full((_D, _D)),
      ],
      out_specs=pl.BlockSpec((1, _D), lambda i: (0, 0)),
      out_shape=jax.ShapeDtypeStruct((1, _D), jnp.float32),
      scratch_shapes=[
          pltpu.VMEM((1, _D), jnp.float32),
          pltpu.VMEM((1, _D), jnp.float32),
      ],
      compiler_params=pltpu.CompilerParams(
          dimension_semantics=("arbitrary",)),
  )(x_pad, acc, inv, cpart, W1l, W1r, b1l, W2l, b2l, W2r)


def kernel(x, edge_index, W1l, b1l, W1r, W2l, b2l, W2r):
  x_pad = jnp.concatenate(
      [x, jnp.zeros((_NP - _N, _D), jnp.float32)], axis=0)
  idx = jnp.arange(_EP - _E, dtype=jnp.int32)
  pad = jnp.stack([_N + idx % 128, _N + idx % (_NPA - _N)])
  ei = jnp.concatenate([edge_index, pad], axis=1)
  src2 = ei[0].reshape(_ROWS, _CHUNK)
  dst2 = ei[1].reshape(_ROWS, _CHUNK)
  ones_h = jnp.ones((_CHUNK,), jnp.float32)
  zrow = jnp.zeros((_NPT,), jnp.float32)
  zacc = jnp.zeros((_NPT, _D), jnp.float32)

  acc, cpart, inv = _sc_graph(x_pad, src2, dst2, ones_h, zrow, zacc)

  out = _tc_dense(x_pad, acc, inv.reshape(_NP, 1),
                  cpart.reshape(_NC, _NP, 1),
                  W1l, W1r, b1l.reshape(1, _D),
                  W2l, b2l.reshape(1, _D), W2r)
  return out.reshape(_D)
